# baseline (speedup 1.0000x reference)
import functools
import math

import jax
import jax.numpy as jnp
from jax import lax
from jax.experimental import pallas as pl
from jax.experimental.pallas import tpu as pltpu
from jax.experimental.pallas import tpu_sc as plsc

_N_POSITIONS = 8192
_D_MODEL = 1024
_N_BITS = math.ceil(math.log2(_N_POSITIONS))
_PAD_BITS = 16
_TBLOCK = 1024

_NW = 32
_TOK = 4 * 8192
_TOK_PER_W = _TOK // _NW
_CH = 32
_NCHUNK = _TOK_PER_W // _CH
_NBUF = 3


def _table_body(emb_ref, t_ref):
    pid = pl.program_id(0)
    row = pid * _TBLOCK + jax.lax.broadcasted_iota(jnp.int32, (_TBLOCK, _PAD_BITS), 0)
    shifts = jax.lax.broadcasted_iota(jnp.int32, (_TBLOCK, _PAD_BITS), 1)
    bits = jnp.bitwise_and(jnp.right_shift(row, shifts), 1)
    t_ref[...] = jnp.dot(bits.astype(jnp.float32), emb_ref[...],
                         preferred_element_type=jnp.float32)


def _build_table(emb):
    embp = jnp.zeros((_PAD_BITS, _D_MODEL), emb.dtype).at[:_N_BITS].set(emb)
    return pl.pallas_call(
        _table_body,
        grid=(_N_POSITIONS // _TBLOCK,),
        in_specs=[pl.BlockSpec((_PAD_BITS, _D_MODEL), lambda i: (0, 0))],
        out_specs=pl.BlockSpec((_TBLOCK, _D_MODEL), lambda i: (i, 0)),
        out_shape=jax.ShapeDtypeStruct((_N_POSITIONS, _D_MODEL), jnp.float32),
    )(embp)


def _sc_body(x_hbm, t_hbm, out_hbm, idx_v, buf0, buf1, buf2,
             gs0, gs1, gs2, ws0, ws1, ws2):
    wid = lax.axis_index("s") * 2 + lax.axis_index("c")
    base = wid * _TOK_PER_W
    pltpu.sync_copy(x_hbm.at[wid], idx_v)
    bufs = (buf0, buf1, buf2)
    gsem = (gs0, gs1, gs2)
    wsem = (ws0, ws1, ws2)
    gcp = [pltpu.async_copy(t_hbm.at[idx_v.at[j]], bufs[j], gsem[j])
           for j in range(_NBUF)]
    wcp = []
    for j in range(_NCHUNK):
        b = j % _NBUF
        gcp[j].wait()
        wcp.append(pltpu.async_copy(
            bufs[b], out_hbm.at[pl.ds(base + j * _CH, _CH)], wsem[b]))
        nj = j + _NBUF
        if nj < _NCHUNK:
            wcp[j].wait()
            gcp.append(pltpu.async_copy(t_hbm.at[idx_v.at[nj]], bufs[b], gsem[b]))
    for j in range(max(0, _NCHUNK - _NBUF), _NCHUNK):
        wcp[j].wait()


_sc_lookup = functools.partial(
    pl.kernel,
    mesh=plsc.VectorSubcoreMesh(core_axis_name="c", subcore_axis_name="s"),
    out_type=jax.ShapeDtypeStruct((_TOK, _D_MODEL), jnp.float32),
    scratch_types=[
        pltpu.VMEM((_NCHUNK, _CH), jnp.int32),
        pltpu.VMEM((_CH, _D_MODEL), jnp.float32),
        pltpu.VMEM((_CH, _D_MODEL), jnp.float32),
        pltpu.VMEM((_CH, _D_MODEL), jnp.float32),
        pltpu.SemaphoreType.DMA,
        pltpu.SemaphoreType.DMA,
        pltpu.SemaphoreType.DMA,
        pltpu.SemaphoreType.DMA,
        pltpu.SemaphoreType.DMA,
        pltpu.SemaphoreType.DMA,
    ],
)(_sc_body)


@jax.jit
def kernel(x, embedding):
    t = _build_table(embedding)
    xw = jnp.reshape(x, (_NW, _NCHUNK, _CH))
    out = _sc_lookup(xw, t)
    return jnp.reshape(out, (*x.shape, _D_MODEL))


# speedup vs baseline: 2.8781x; 2.8781x over previous
import functools
import math

import jax
import jax.numpy as jnp
from jax.experimental import pallas as pl
from jax.experimental.pallas import tpu as pltpu

_N_POSITIONS = 8192
_D_MODEL = 1024
_N_BITS = math.ceil(math.log2(_N_POSITIONS))
_PAD_BITS = 16
_BLOCK = 2048


def _body(x_ref, emb_ref, o_ref):
    xb = x_ref[0, 0, :]
    shifts = jax.lax.broadcasted_iota(jnp.int32, (_BLOCK, _N_BITS), 1)
    bits = jnp.bitwise_and(jnp.right_shift(xb[:, None], shifts), 1)
    o_ref[0] = jnp.dot(bits.astype(jnp.float32), emb_ref[...],
                       preferred_element_type=jnp.float32)


@jax.jit
def kernel(x, embedding):
    n_tokens = x.size
    nb = n_tokens // _BLOCK
    x3 = jnp.reshape(x, (nb, 1, _BLOCK))
    out = pl.pallas_call(
        _body,
        grid=(nb,),
        in_specs=[
            pl.BlockSpec((1, 1, _BLOCK), lambda i: (i, 0, 0)),
            pl.BlockSpec((_N_BITS, _D_MODEL), lambda i: (0, 0)),
        ],
        out_specs=pl.BlockSpec((1, _BLOCK, _D_MODEL), lambda i: (i, 0, 0)),
        out_shape=jax.ShapeDtypeStruct((nb, _BLOCK, _D_MODEL), jnp.float32),
        compiler_params=pltpu.CompilerParams(
            dimension_semantics=("arbitrary",)),
    )(x3, embedding)
    return jnp.reshape(out, (*x.shape, _D_MODEL))


# speedup vs baseline: 2.9817x; 1.0360x over previous
import functools
import math

import jax
import jax.numpy as jnp
from jax.experimental import pallas as pl
from jax.experimental.pallas import tpu as pltpu

_N_POSITIONS = 8192
_D_MODEL = 1024
_N_BITS = math.ceil(math.log2(_N_POSITIONS))
_PAD_BITS = 16
_BLOCK = 2048


def _body(x_ref, emb_ref, o_ref):
    i = pl.program_id(0)
    xb = x_ref[i // 4, pl.ds((i % 4) * _BLOCK, _BLOCK)]
    shifts = jax.lax.broadcasted_iota(jnp.int32, (_BLOCK, _N_BITS), 1)
    bits = jnp.bitwise_and(jnp.right_shift(xb[:, None], shifts), 1)
    o_ref[0] = jnp.dot(bits.astype(jnp.float32), emb_ref[...],
                       preferred_element_type=jnp.float32)


@jax.jit
def kernel(x, embedding):
    n_tokens = x.size
    nb = n_tokens // _BLOCK
    out = pl.pallas_call(
        _body,
        grid=(nb,),
        in_specs=[
            pl.BlockSpec((4, 8192), lambda i: (0, 0)),
            pl.BlockSpec((_N_BITS, _D_MODEL), lambda i: (0, 0)),
        ],
        out_specs=pl.BlockSpec((1, _BLOCK, _D_MODEL), lambda i: (i, 0, 0)),
        out_shape=jax.ShapeDtypeStruct((nb, _BLOCK, _D_MODEL), jnp.float32),
        compiler_params=pltpu.CompilerParams(
            dimension_semantics=("arbitrary",)),
    )(x, embedding)
    return jnp.reshape(out, (*x.shape, _D_MODEL))
